# Initial kernel scaffold; baseline (speedup 1.0000x reference)
import jax
import jax.numpy as jnp
from jax.experimental import pallas as pl


def kernel(x_node, x_edge, edge_index, W, b):
    raise NotImplementedError("write your pallas kernel here")


# speedup vs baseline: 7.3299x; 7.3299x over previous
import functools

import jax
import jax.numpy as jnp
from jax import lax
from jax.experimental import pallas as pl
from jax.experimental.pallas import tpu as pltpu
from jax.experimental.pallas import tpu_sc as plsc

N = 10000
E = 320000
D_NODE = 128
D_EDGE = 16

NC = 2
NS = 16
NW = NC * NS

EPT = E // NW
SUB = 100
NCHUNK = EPT // SUB
CH = 2000
NSUPER = EPT // CH
NSUB = CH // SUB
ZROWS = 624
TAIL = N - NS * ZROWS

_mesh = plsc.VectorSubcoreMesh(core_axis_name="c", subcore_axis_name="s")


@functools.partial(
    pl.kernel,
    out_type=jax.ShapeDtypeStruct((NC, N, D_EDGE), jnp.float32),
    mesh=_mesh,
    scratch_types=[
        pltpu.VMEM((NCHUNK, SUB), jnp.int32),
        pltpu.VMEM((NCHUNK, SUB), jnp.int32),
        pltpu.VMEM((CH, D_EDGE), jnp.float32),
        pltpu.VMEM_SHARED((N, D_EDGE), jnp.float32),
    ],
    compiler_params=pltpu.CompilerParams(use_tc_tiling_on_sc=False),
)
def _scatter_add_sc(eidx_hbm, xedge_hbm, zeros_hbm, out_hbm,
                    idx_s, idx_d, ebuf, acc):
    c = lax.axis_index("c")
    s = lax.axis_index("s")
    t = s * NC + c
    base = t * EPT
    pltpu.sync_copy(zeros_hbm, acc.at[pl.ds(s * ZROWS, ZROWS)])

    @pl.when(s == 0)
    def _zero_tail():
        pltpu.sync_copy(zeros_hbm.at[pl.ds(0, TAIL)],
                        acc.at[pl.ds(NS * ZROWS, TAIL)])
    pltpu.sync_copy(eidx_hbm.at[0, t], idx_s)
    pltpu.sync_copy(eidx_hbm.at[1, t], idx_d)
    plsc.subcore_barrier()
    for sup in range(NSUPER):
        pltpu.sync_copy(xedge_hbm.at[pl.ds(base + sup * CH, CH)], ebuf)

        def body(j, _, sup=sup):
            r = sup * NSUB + j
            rows = ebuf.at[pl.ds(j * SUB, SUB)]
            pltpu.sync_copy(rows, acc.at[idx_s.at[r]], add=True)
            pltpu.sync_copy(rows, acc.at[idx_d.at[r]], add=True)
            return 0

        lax.fori_loop(0, NSUB, body, 0)
    plsc.subcore_barrier()
    pltpu.sync_copy(acc.at[pl.ds(s * ZROWS, ZROWS)],
                    out_hbm.at[c, pl.ds(s * ZROWS, ZROWS)])

    @pl.when(s == 0)
    def _flush_tail():
        pltpu.sync_copy(acc.at[pl.ds(NS * ZROWS, TAIL)],
                        out_hbm.at[c, pl.ds(NS * ZROWS, TAIL)])


def _linear_body(x_ref, p0_ref, p1_ref, w1_ref, w2_ref, b_ref, o_ref):
    pb = p0_ref[...] + p1_ref[...]
    o_ref[...] = (
        jnp.dot(x_ref[...], w1_ref[...], preferred_element_type=jnp.float32)
        + jnp.dot(pb, w2_ref[...], preferred_element_type=jnp.float32)
        + b_ref[...]
    )


_BM = 1000


def _linear_tc(x_node, p0, p1, W1, W2, b2d):
    return pl.pallas_call(
        _linear_body,
        grid=(N // _BM,),
        in_specs=[
            pl.BlockSpec((_BM, D_NODE), lambda i: (i, 0)),
            pl.BlockSpec((_BM, D_EDGE), lambda i: (i, 0)),
            pl.BlockSpec((_BM, D_EDGE), lambda i: (i, 0)),
            pl.BlockSpec((D_NODE, D_NODE), lambda i: (0, 0)),
            pl.BlockSpec((D_EDGE, D_NODE), lambda i: (0, 0)),
            pl.BlockSpec((1, D_NODE), lambda i: (0, 0)),
        ],
        out_specs=pl.BlockSpec((_BM, D_NODE), lambda i: (i, 0)),
        out_shape=jax.ShapeDtypeStruct((N, D_NODE), jnp.float32),
    )(x_node, p0, p1, W1, W2, b2d)


def kernel(x_node, x_edge, edge_index, W, b):
    eidx = edge_index.reshape(2, NW, NCHUNK, SUB)
    zeros = jnp.zeros((ZROWS, D_EDGE), jnp.float32)
    partials = _scatter_add_sc(eidx, x_edge, zeros)
    x_node_out = _linear_tc(
        x_node, partials[0], partials[1],
        W[:D_NODE], W[D_NODE:], b.reshape(1, D_NODE),
    )
    return (x_node_out, x_edge, edge_index)


# speedup vs baseline: 7.4983x; 1.0230x over previous
import functools

import jax
import jax.numpy as jnp
from jax import lax
from jax.experimental import pallas as pl
from jax.experimental.pallas import tpu as pltpu
from jax.experimental.pallas import tpu_sc as plsc

N = 10000
E = 320000
D_NODE = 128
D_EDGE = 16

NC = 2
NS = 16
NW = NC * NS

EPT = E // NW
SUB = 80
CH = 2000
NSUPER = EPT // CH
NSUB = CH // SUB
ZROWS = 624
TAIL = N - NS * ZROWS

_mesh = plsc.VectorSubcoreMesh(core_axis_name="c", subcore_axis_name="s")


@functools.partial(
    pl.kernel,
    out_type=jax.ShapeDtypeStruct((NC, N, D_EDGE), jnp.float32),
    mesh=_mesh,
    scratch_types=[
        pltpu.VMEM((EPT,), jnp.int32),
        pltpu.VMEM((EPT,), jnp.int32),
        pltpu.VMEM((CH, D_EDGE), jnp.float32),
        pltpu.VMEM_SHARED((N, D_EDGE), jnp.float32),
    ],
    compiler_params=pltpu.CompilerParams(use_tc_tiling_on_sc=False),
)
def _scatter_add_sc(eidx_hbm, xedge_hbm, zeros_hbm, out_hbm,
                    idx_s, idx_d, ebuf, acc):
    c = lax.axis_index("c")
    s = lax.axis_index("s")
    t = s * NC + c
    base = t * EPT
    pltpu.sync_copy(zeros_hbm, acc.at[pl.ds(s * ZROWS, ZROWS)])

    @pl.when(s == 0)
    def _zero_tail():
        pltpu.sync_copy(zeros_hbm.at[pl.ds(0, TAIL)],
                        acc.at[pl.ds(NS * ZROWS, TAIL)])
    pltpu.sync_copy(eidx_hbm.at[0, pl.ds(base, EPT)], idx_s)
    pltpu.sync_copy(eidx_hbm.at[1, pl.ds(base, EPT)], idx_d)
    plsc.subcore_barrier()
    for sup in range(NSUPER):
        pltpu.sync_copy(xedge_hbm.at[pl.ds(base + sup * CH, CH)], ebuf)

        def body(j, _, sup=sup):
            r = sup * CH + j * SUB
            rows = ebuf.at[pl.ds(j * SUB, SUB)]
            pltpu.sync_copy(rows, acc.at[idx_s.at[pl.ds(r, SUB)]], add=True)
            pltpu.sync_copy(rows, acc.at[idx_d.at[pl.ds(r, SUB)]], add=True)
            return 0

        lax.fori_loop(0, NSUB, body, 0)
    plsc.subcore_barrier()
    pltpu.sync_copy(acc.at[pl.ds(s * ZROWS, ZROWS)],
                    out_hbm.at[c, pl.ds(s * ZROWS, ZROWS)])

    @pl.when(s == 0)
    def _flush_tail():
        pltpu.sync_copy(acc.at[pl.ds(NS * ZROWS, TAIL)],
                        out_hbm.at[c, pl.ds(NS * ZROWS, TAIL)])


def _linear_body(x_ref, p_ref, w1_ref, w2_ref, b_ref, o_ref):
    pb = p_ref[0] + p_ref[1]
    o_ref[...] = (
        jnp.dot(x_ref[...], w1_ref[...], preferred_element_type=jnp.float32)
        + jnp.dot(pb, w2_ref[...], preferred_element_type=jnp.float32)
        + b_ref[...]
    )


_BM = 1000


def _linear_tc(x_node, partials, W1, W2, b2d):
    return pl.pallas_call(
        _linear_body,
        grid=(N // _BM,),
        in_specs=[
            pl.BlockSpec((_BM, D_NODE), lambda i: (i, 0)),
            pl.BlockSpec((2, _BM, D_EDGE), lambda i: (0, i, 0)),
            pl.BlockSpec((D_NODE, D_NODE), lambda i: (0, 0)),
            pl.BlockSpec((D_EDGE, D_NODE), lambda i: (0, 0)),
            pl.BlockSpec((1, D_NODE), lambda i: (0, 0)),
        ],
        out_specs=pl.BlockSpec((_BM, D_NODE), lambda i: (i, 0)),
        out_shape=jax.ShapeDtypeStruct((N, D_NODE), jnp.float32),
    )(x_node, partials, W1, W2, b2d)


def kernel(x_node, x_edge, edge_index, W, b):
    zeros = jnp.zeros((ZROWS, D_EDGE), jnp.float32)
    partials = _scatter_add_sc(edge_index, x_edge, zeros)
    x_node_out = _linear_tc(
        x_node, partials,
        W[:D_NODE], W[D_NODE:], b.reshape(1, D_NODE),
    )
    return (x_node_out, x_edge, edge_index)


# speedup vs baseline: 10.7624x; 1.4353x over previous
import functools

import jax
import jax.numpy as jnp
from jax import lax
from jax.experimental import pallas as pl
from jax.experimental.pallas import tpu as pltpu
from jax.experimental.pallas import tpu_sc as plsc

N = 10000
E = 320000
D_NODE = 128
D_EDGE = 16

NC = 2
NS = 16
NW = NC * NS

BLK = 128
NB = E // BLK
NBT = NB // NW
NTAIL = NB - NBT * NW
CB = 13
NSUP = NBT // CB
CHW = CB * 8 * BLK
IDXN = NBT * BLK

ZROWS = 624
TAIL = N - NS * ZROWS

_mesh = plsc.VectorSubcoreMesh(core_axis_name="c", subcore_axis_name="s")


@functools.partial(
    pl.kernel,
    out_type=jax.ShapeDtypeStruct((NC, N, D_EDGE), jnp.float32),
    mesh=_mesh,
    scratch_types=[
        pltpu.VMEM((IDXN + BLK,), jnp.int32),
        pltpu.VMEM((IDXN + BLK,), jnp.int32),
        pltpu.VMEM((2 * CHW,), jnp.float32),
        pltpu.VMEM((CB * BLK, D_EDGE), jnp.float32),
        pltpu.VMEM_SHARED((N, D_EDGE), jnp.float32),
    ],
    compiler_params=pltpu.CompilerParams(use_tc_tiling_on_sc=False,
                                         needs_layout_passes=False),
)
def _scatter_add_sc(eidx_hbm, z_hbm, zeros_hbm, out_hbm,
                    idx_s, idx_d, chunk, rowbuf, acc):
    c = lax.axis_index("c")
    s = lax.axis_index("s")
    t = s * NC + c
    estart = t * IDXN
    pltpu.sync_copy(zeros_hbm, acc.at[pl.ds(s * ZROWS, ZROWS)])

    @pl.when(s == 0)
    def _zero_tail():
        pltpu.sync_copy(zeros_hbm.at[pl.ds(0, TAIL)],
                        acc.at[pl.ds(NS * ZROWS, TAIL)])

    pltpu.sync_copy(eidx_hbm.at[0, pl.ds(estart, IDXN)],
                    idx_s.at[pl.ds(0, IDXN)])
    pltpu.sync_copy(eidx_hbm.at[1, pl.ds(estart, IDXN)],
                    idx_d.at[pl.ds(0, IDXN)])

    @pl.when(t < NTAIL)
    def _tail_idx():
        tstart = NW * IDXN + t * BLK
        pltpu.sync_copy(eidx_hbm.at[0, pl.ds(tstart, BLK)],
                        idx_s.at[pl.ds(IDXN, BLK)])
        pltpu.sync_copy(eidx_hbm.at[1, pl.ds(tstart, BLK)],
                        idx_d.at[pl.ds(IDXN, BLK)])

    plsc.subcore_barrier()
    iota16 = lax.iota(jnp.int32, 16)

    def transpose_block(j):
        rows = [iota16 + (j * BLK + 16 * lg) for lg in range(8)]
        for k in range(D_EDGE):
            tr, rr = divmod(k, 8)
            col = jnp.full((16,), k, jnp.int32)
            off0 = tr * CHW + rr * BLK
            for lg in range(8):
                v = chunk[pl.ds(off0 + j * (8 * BLK) + 16 * lg, 16)]
                plsc.store_scatter(rowbuf, [rows[lg], col], v)

    def scatter_rows(m, ioff):
        rws = rowbuf.at[pl.ds(m * BLK, BLK)]
        pltpu.sync_copy(rws, acc.at[idx_s.at[pl.ds(ioff, BLK)]], add=True)
        pltpu.sync_copy(rws, acc.at[idx_d.at[pl.ds(ioff, BLK)]], add=True)

    for sup in range(NSUP):
        zoff = (t * NBT + sup * CB) * (8 * BLK)
        pltpu.sync_copy(z_hbm.at[0, pl.ds(zoff, CHW)], chunk.at[pl.ds(0, CHW)])
        pltpu.sync_copy(z_hbm.at[1, pl.ds(zoff, CHW)],
                        chunk.at[pl.ds(CHW, CHW)])

        def jbody(j, _):
            transpose_block(j)
            return 0

        lax.fori_loop(0, CB, jbody, 0)

        def mbody(m, _, sup=sup):
            scatter_rows(m, sup * CB * BLK + m * BLK)
            return 0

        lax.fori_loop(0, CB, mbody, 0)

    @pl.when(t < NTAIL)
    def _tail_block():
        zoff = (NW * NBT + t) * (8 * BLK)
        pltpu.sync_copy(z_hbm.at[0, pl.ds(zoff, 8 * BLK)],
                        chunk.at[pl.ds(0, 8 * BLK)])
        pltpu.sync_copy(z_hbm.at[1, pl.ds(zoff, 8 * BLK)],
                        chunk.at[pl.ds(CHW, 8 * BLK)])
        transpose_block(0)
        scatter_rows(0, IDXN)

    plsc.subcore_barrier()
    pltpu.sync_copy(acc.at[pl.ds(s * ZROWS, ZROWS)],
                    out_hbm.at[c, pl.ds(s * ZROWS, ZROWS)])

    @pl.when(s == 0)
    def _flush_tail():
        pltpu.sync_copy(acc.at[pl.ds(NS * ZROWS, TAIL)],
                        out_hbm.at[c, pl.ds(NS * ZROWS, TAIL)])


def _linear_body(x_ref, p_ref, w1_ref, w2_ref, b_ref, o_ref):
    pb = p_ref[0] + p_ref[1]
    o_ref[...] = (
        jnp.dot(x_ref[...], w1_ref[...], preferred_element_type=jnp.float32)
        + jnp.dot(pb, w2_ref[...], preferred_element_type=jnp.float32)
        + b_ref[...]
    )


_BM = 1000


def _linear_tc(x_node, partials, W1, W2, b2d):
    return pl.pallas_call(
        _linear_body,
        grid=(N // _BM,),
        in_specs=[
            pl.BlockSpec((_BM, D_NODE), lambda i: (i, 0)),
            pl.BlockSpec((2, _BM, D_EDGE), lambda i: (0, i, 0)),
            pl.BlockSpec((D_NODE, D_NODE), lambda i: (0, 0)),
            pl.BlockSpec((D_EDGE, D_NODE), lambda i: (0, 0)),
            pl.BlockSpec((1, D_NODE), lambda i: (0, 0)),
        ],
        out_specs=pl.BlockSpec((_BM, D_NODE), lambda i: (i, 0)),
        out_shape=jax.ShapeDtypeStruct((N, D_NODE), jnp.float32),
    )(x_node, partials, W1, W2, b2d)


def kernel(x_node, x_edge, edge_index, W, b):
    z = (x_edge.T.reshape(2, 8, NB, BLK)
         .transpose(0, 2, 1, 3)
         .reshape(2, NB * 8 * BLK))
    zeros = jnp.zeros((ZROWS, D_EDGE), jnp.float32)
    partials = _scatter_add_sc(edge_index, z, zeros)
    x_node_out = _linear_tc(
        x_node, partials,
        W[:D_NODE], W[D_NODE:], b.reshape(1, D_NODE),
    )
    return (x_node_out, x_edge, edge_index)


# speedup vs baseline: 11.3965x; 1.0589x over previous
import functools

import jax
import jax.numpy as jnp
from jax import lax
from jax.experimental import pallas as pl
from jax.experimental.pallas import tpu as pltpu
from jax.experimental.pallas import tpu_sc as plsc

N = 10000
E = 320000
D_NODE = 128
D_EDGE = 16

NC = 2
NS = 16
NW = NC * NS

BLK = 128
NB = E // BLK
NBT = NB // NW
NTAIL = NB - NBT * NW
CB = 13
NSUP = NBT // CB
CHW = CB * 8 * BLK
IDXN = NBT * BLK

ZROWS = 624
TAIL = N - NS * ZROWS

_mesh = plsc.VectorSubcoreMesh(core_axis_name="c", subcore_axis_name="s")


@functools.partial(
    pl.kernel,
    out_type=jax.ShapeDtypeStruct((NC, N, D_EDGE), jnp.float32),
    mesh=_mesh,
    scratch_types=[
        pltpu.VMEM((IDXN + BLK,), jnp.int32),
        pltpu.VMEM((IDXN + BLK,), jnp.int32),
        pltpu.VMEM((2 * CHW,), jnp.float32),
        pltpu.VMEM((CB * BLK, D_EDGE), jnp.float32),
        pltpu.VMEM_SHARED((N, D_EDGE), jnp.float32),
    ],
    compiler_params=pltpu.CompilerParams(use_tc_tiling_on_sc=False,
                                         needs_layout_passes=False),
)
def _scatter_add_sc(eidx_hbm, z_hbm, zeros_hbm, out_hbm,
                    idx_s, idx_d, chunk, rowbuf, acc):
    c = lax.axis_index("c")
    s = lax.axis_index("s")
    t = s * NC + c
    estart = t * IDXN
    pltpu.sync_copy(zeros_hbm, acc.at[pl.ds(s * ZROWS, ZROWS)])

    @pl.when(s == 0)
    def _zero_tail():
        pltpu.sync_copy(zeros_hbm.at[pl.ds(0, TAIL)],
                        acc.at[pl.ds(NS * ZROWS, TAIL)])

    pltpu.sync_copy(eidx_hbm.at[0, pl.ds(estart, IDXN)],
                    idx_s.at[pl.ds(0, IDXN)])
    pltpu.sync_copy(eidx_hbm.at[1, pl.ds(estart, IDXN)],
                    idx_d.at[pl.ds(0, IDXN)])

    @pl.when(t < NTAIL)
    def _tail_idx():
        tstart = NW * IDXN + t * BLK
        pltpu.sync_copy(eidx_hbm.at[0, pl.ds(tstart, BLK)],
                        idx_s.at[pl.ds(IDXN, BLK)])
        pltpu.sync_copy(eidx_hbm.at[1, pl.ds(tstart, BLK)],
                        idx_d.at[pl.ds(IDXN, BLK)])

    plsc.subcore_barrier()
    iota16 = lax.iota(jnp.int32, 16)

    def transpose_block(j):
        rows = [iota16 + (j * BLK + 16 * lg) for lg in range(8)]
        for k in range(D_EDGE):
            tr, rr = divmod(k, 8)
            col = jnp.full((16,), k, jnp.int32)
            off0 = tr * CHW + rr * BLK
            for lg in range(8):
                v = chunk[pl.ds(off0 + j * (8 * BLK) + 16 * lg, 16)]
                plsc.store_scatter(rowbuf, [rows[lg], col], v)

    def scatter_rows(m, ioff):
        rws = rowbuf.at[pl.ds(m * BLK, BLK)]
        pltpu.sync_copy(rws, acc.at[idx_s.at[pl.ds(ioff, BLK)]], add=True)
        pltpu.sync_copy(rws, acc.at[idx_d.at[pl.ds(ioff, BLK)]], add=True)

    for sup in range(NSUP):
        zoff = (t * NBT + sup * CB) * (8 * BLK)
        pltpu.sync_copy(z_hbm.at[0, pl.ds(zoff, CHW)], chunk.at[pl.ds(0, CHW)])
        pltpu.sync_copy(z_hbm.at[1, pl.ds(zoff, CHW)],
                        chunk.at[pl.ds(CHW, CHW)])

        def jbody(j, _):
            transpose_block(j)
            return 0

        lax.fori_loop(0, CB, jbody, 0)

        ioff = sup * CB * BLK
        pltpu.sync_copy(rowbuf, acc.at[idx_s.at[pl.ds(ioff, CB * BLK)]],
                        add=True)
        pltpu.sync_copy(rowbuf, acc.at[idx_d.at[pl.ds(ioff, CB * BLK)]],
                        add=True)

    @pl.when(t < NTAIL)
    def _tail_block():
        zoff = (NW * NBT + t) * (8 * BLK)
        pltpu.sync_copy(z_hbm.at[0, pl.ds(zoff, 8 * BLK)],
                        chunk.at[pl.ds(0, 8 * BLK)])
        pltpu.sync_copy(z_hbm.at[1, pl.ds(zoff, 8 * BLK)],
                        chunk.at[pl.ds(CHW, 8 * BLK)])
        transpose_block(0)
        scatter_rows(0, IDXN)

    plsc.subcore_barrier()
    pltpu.sync_copy(acc.at[pl.ds(s * ZROWS, ZROWS)],
                    out_hbm.at[c, pl.ds(s * ZROWS, ZROWS)])

    @pl.when(s == 0)
    def _flush_tail():
        pltpu.sync_copy(acc.at[pl.ds(NS * ZROWS, TAIL)],
                        out_hbm.at[c, pl.ds(NS * ZROWS, TAIL)])


def _linear_body(x_ref, p_ref, w1_ref, w2_ref, b_ref, o_ref):
    pb = p_ref[0] + p_ref[1]
    o_ref[...] = (
        jnp.dot(x_ref[...], w1_ref[...], preferred_element_type=jnp.float32)
        + jnp.dot(pb, w2_ref[...], preferred_element_type=jnp.float32)
        + b_ref[...]
    )


_BM = 1000


def _linear_tc(x_node, partials, W1, W2, b2d):
    return pl.pallas_call(
        _linear_body,
        grid=(N // _BM,),
        in_specs=[
            pl.BlockSpec((_BM, D_NODE), lambda i: (i, 0)),
            pl.BlockSpec((2, _BM, D_EDGE), lambda i: (0, i, 0)),
            pl.BlockSpec((D_NODE, D_NODE), lambda i: (0, 0)),
            pl.BlockSpec((D_EDGE, D_NODE), lambda i: (0, 0)),
            pl.BlockSpec((1, D_NODE), lambda i: (0, 0)),
        ],
        out_specs=pl.BlockSpec((_BM, D_NODE), lambda i: (i, 0)),
        out_shape=jax.ShapeDtypeStruct((N, D_NODE), jnp.float32),
    )(x_node, partials, W1, W2, b2d)


def kernel(x_node, x_edge, edge_index, W, b):
    z = (x_edge.T.reshape(2, 8, NB, BLK)
         .transpose(0, 2, 1, 3)
         .reshape(2, NB * 8 * BLK))
    zeros = jnp.zeros((ZROWS, D_EDGE), jnp.float32)
    partials = _scatter_add_sc(edge_index, z, zeros)
    x_node_out = _linear_tc(
        x_node, partials,
        W[:D_NODE], W[D_NODE:], b.reshape(1, D_NODE),
    )
    return (x_node_out, x_edge, edge_index)


# speedup vs baseline: 13.7811x; 1.2092x over previous
import functools

import jax
import jax.numpy as jnp
from jax import lax
from jax.experimental import pallas as pl
from jax.experimental.pallas import tpu as pltpu
from jax.experimental.pallas import tpu_sc as plsc

N = 10000
E = 320000
D_NODE = 128
D_EDGE = 16

NC = 2
NS = 16
NW = NC * NS

BLK = 128
NB = E // BLK
NBT = NB // NW
NTAIL = NB - NBT * NW
CB = 13
NSUP = NBT // CB
CHW = CB * 8 * BLK
IDXN = NBT * BLK

ZROWS = 624
TAIL = N - NS * ZROWS

_mesh = plsc.VectorSubcoreMesh(core_axis_name="c", subcore_axis_name="s")


@functools.partial(
    pl.kernel,
    out_type=jax.ShapeDtypeStruct((NC, N, D_EDGE), jnp.float32),
    mesh=_mesh,
    scratch_types=[
        pltpu.VMEM((IDXN + BLK,), jnp.int32),
        pltpu.VMEM((IDXN + BLK,), jnp.int32),
        pltpu.VMEM((2 * CHW,), jnp.float32),
        pltpu.VMEM((CB * BLK, D_EDGE), jnp.float32),
        pltpu.VMEM_SHARED((N, D_EDGE), jnp.float32),
    ],
    compiler_params=pltpu.CompilerParams(use_tc_tiling_on_sc=False,
                                         needs_layout_passes=False),
)
def _scatter_add_sc(eidx_hbm, z_hbm, zeros_hbm, out_hbm,
                    idx_s, idx_d, chunk, rowbuf, acc):
    c = lax.axis_index("c")
    s = lax.axis_index("s")
    t = s * NC + c
    estart = t * IDXN
    pltpu.sync_copy(zeros_hbm, acc.at[pl.ds(s * ZROWS, ZROWS)])

    @pl.when(s == 0)
    def _zero_tail():
        pltpu.sync_copy(zeros_hbm.at[pl.ds(0, TAIL)],
                        acc.at[pl.ds(NS * ZROWS, TAIL)])

    pltpu.sync_copy(eidx_hbm.at[0, pl.ds(estart, IDXN)],
                    idx_s.at[pl.ds(0, IDXN)])
    pltpu.sync_copy(eidx_hbm.at[1, pl.ds(estart, IDXN)],
                    idx_d.at[pl.ds(0, IDXN)])

    @pl.when(t < NTAIL)
    def _tail_idx():
        tstart = NW * IDXN + t * BLK
        pltpu.sync_copy(eidx_hbm.at[0, pl.ds(tstart, BLK)],
                        idx_s.at[pl.ds(IDXN, BLK)])
        pltpu.sync_copy(eidx_hbm.at[1, pl.ds(tstart, BLK)],
                        idx_d.at[pl.ds(IDXN, BLK)])

    plsc.subcore_barrier()
    iota16 = lax.iota(jnp.int32, 16)

    def transpose_block(j):
        rows = [iota16 + (j * BLK + 16 * lg) for lg in range(8)]
        for k in range(D_EDGE):
            tr, rr = divmod(k, 8)
            col = jnp.full((16,), k, jnp.int32)
            off0 = tr * CHW + rr * BLK
            for lg in range(8):
                v = chunk[pl.ds(off0 + j * (8 * BLK) + 16 * lg, 16)]
                plsc.store_scatter(rowbuf, [rows[lg], col], v)

    def scatter_rows(m, ioff):
        rws = rowbuf.at[pl.ds(m * BLK, BLK)]
        pltpu.sync_copy(rws, acc.at[idx_s.at[pl.ds(ioff, BLK)]], add=True)
        pltpu.sync_copy(rws, acc.at[idx_d.at[pl.ds(ioff, BLK)]], add=True)

    def sup_body(sup, _):
        zoff = (t * NBT + sup * CB) * (8 * BLK)
        pltpu.sync_copy(z_hbm.at[0, pl.ds(zoff, CHW)], chunk.at[pl.ds(0, CHW)])
        pltpu.sync_copy(z_hbm.at[1, pl.ds(zoff, CHW)],
                        chunk.at[pl.ds(CHW, CHW)])

        @plsc.parallel_loop(0, CB, 1, unroll=2)
        def _transpose_loop(j):
            transpose_block(j)

        ioff = sup * CB * BLK
        pltpu.sync_copy(rowbuf, acc.at[idx_s.at[pl.ds(ioff, CB * BLK)]],
                        add=True)
        pltpu.sync_copy(rowbuf, acc.at[idx_d.at[pl.ds(ioff, CB * BLK)]],
                        add=True)
        return 0

    lax.fori_loop(0, NSUP, sup_body, 0)

    @pl.when(t < NTAIL)
    def _tail_block():
        zoff = (NW * NBT + t) * (8 * BLK)
        pltpu.sync_copy(z_hbm.at[0, pl.ds(zoff, 8 * BLK)],
                        chunk.at[pl.ds(0, 8 * BLK)])
        pltpu.sync_copy(z_hbm.at[1, pl.ds(zoff, 8 * BLK)],
                        chunk.at[pl.ds(CHW, 8 * BLK)])
        transpose_block(0)
        scatter_rows(0, IDXN)

    plsc.subcore_barrier()
    pltpu.sync_copy(acc.at[pl.ds(s * ZROWS, ZROWS)],
                    out_hbm.at[c, pl.ds(s * ZROWS, ZROWS)])

    @pl.when(s == 0)
    def _flush_tail():
        pltpu.sync_copy(acc.at[pl.ds(NS * ZROWS, TAIL)],
                        out_hbm.at[c, pl.ds(NS * ZROWS, TAIL)])


def _linear_body(x_ref, p_ref, w1_ref, w2_ref, b_ref, o_ref):
    pb = p_ref[0] + p_ref[1]
    o_ref[...] = (
        jnp.dot(x_ref[...], w1_ref[...], preferred_element_type=jnp.float32)
        + jnp.dot(pb, w2_ref[...], preferred_element_type=jnp.float32)
        + b_ref[...]
    )


_BM = 1000


def _linear_tc(x_node, partials, W1, W2, b2d):
    return pl.pallas_call(
        _linear_body,
        grid=(N // _BM,),
        in_specs=[
            pl.BlockSpec((_BM, D_NODE), lambda i: (i, 0)),
            pl.BlockSpec((2, _BM, D_EDGE), lambda i: (0, i, 0)),
            pl.BlockSpec((D_NODE, D_NODE), lambda i: (0, 0)),
            pl.BlockSpec((D_EDGE, D_NODE), lambda i: (0, 0)),
            pl.BlockSpec((1, D_NODE), lambda i: (0, 0)),
        ],
        out_specs=pl.BlockSpec((_BM, D_NODE), lambda i: (i, 0)),
        out_shape=jax.ShapeDtypeStruct((N, D_NODE), jnp.float32),
    )(x_node, partials, W1, W2, b2d)


def kernel(x_node, x_edge, edge_index, W, b):
    z = (x_edge.T.reshape(2, 8, NB, BLK)
         .transpose(0, 2, 1, 3)
         .reshape(2, NB * 8 * BLK))
    zeros = jnp.zeros((ZROWS, D_EDGE), jnp.float32)
    partials = _scatter_add_sc(edge_index, z, zeros)
    x_node_out = _linear_tc(
        x_node, partials,
        W[:D_NODE], W[D_NODE:], b.reshape(1, D_NODE),
    )
    return (x_node_out, x_edge, edge_index)
